# baseline (speedup 1.0000x reference)
import functools

import jax
import jax.numpy as jnp
from jax.experimental import pallas as pl
from jax.experimental.pallas import tpu as pltpu

B, N, D = 2, 4096, 1024
S, K, H = 64, 2, 512

CN = 1024
NB = N // CN


def _route_kernel(x_ref, wg_ref, bg_ref, comp_ref, cnt_ref):
    nb = pl.program_id(1)
    xc = x_ref[0]
    logits = jax.lax.dot_general(
        xc, wg_ref[...], (((1,), (0,)), ((), ())),
        preferred_element_type=jnp.float32) + bg_ref[...]
    m1 = jnp.max(logits, axis=-1, keepdims=True)
    l2 = jnp.where(logits >= m1, -jnp.inf, logits)
    m2 = jnp.max(l2, axis=-1, keepdims=True)
    e = jnp.exp(logits - m1)
    sum_e = jnp.sum(e, axis=-1, keepdims=True)
    w = jnp.where(logits >= m2, e, 0.0) / sum_e

    part = jax.lax.dot_general(
        w, xc, (((0,), (0,)), ((), ())),
        preferred_element_type=jnp.float32)
    csum = jnp.sum(w, axis=0)[None, :]

    @pl.when(nb == 0)
    def _init():
        comp_ref[0] = part
        cnt_ref[...] = csum

    @pl.when(nb > 0)
    def _acc():
        comp_ref[0] += part
        cnt_ref[...] += csum

    @pl.when(nb == NB - 1)
    def _final():
        comp_ref[0] = comp_ref[0] / (cnt_ref[0][:, None] + 1e-9)


ST = 2
NSTEP = S // ST
NBUF = 4


def _mlp_kernel(comp_ref, b1_ref, b2_ref, g_ref, be_ref, w1_hbm, w2_hbm,
                out_ref, w1buf, w2buf, sems):
    i = pl.program_id(0)

    @pl.when(i == 0)
    def _prologue():
        for k in range(NBUF):
            pltpu.make_async_copy(
                w1_hbm.at[pl.ds(k * ST, ST)], w1buf.at[k],
                sems.at[k, 0]).start()
            pltpu.make_async_copy(
                w2_hbm.at[pl.ds(k * ST, ST)], w2buf.at[k],
                sems.at[k, 1]).start()

    slot = jax.lax.rem(i, NBUF)
    pltpu.make_async_copy(
        w1_hbm.at[pl.ds(i * ST, ST)], w1buf.at[slot], sems.at[slot, 0]).wait()
    pltpu.make_async_copy(
        w2_hbm.at[pl.ds(i * ST, ST)], w2buf.at[slot], sems.at[slot, 1]).wait()

    for j in range(ST):
        c = comp_ref[i * ST + j]
        h = jax.lax.dot_general(
            c, w1buf[slot, j], (((1,), (0,)), ((), ())),
            preferred_element_type=jnp.float32) + b1_ref[i * ST + j]
        h = 0.5 * h * (1.0 + jax.lax.erf(h * 0.7071067811865476))
        y = jax.lax.dot_general(
            h, w2buf[slot, j], (((1,), (0,)), ((), ())),
            preferred_element_type=jnp.float32) + b2_ref[i * ST + j]
        mu = jnp.mean(y, axis=-1, keepdims=True)
        var = jnp.mean((y - mu) ** 2, axis=-1, keepdims=True)
        ln = ((y - mu) * jax.lax.rsqrt(var + 1e-5) * g_ref[i * ST + j]
              + be_ref[i * ST + j])
        out_ref[j] = c + ln

    nxt = i + NBUF

    @pl.when(nxt < NSTEP)
    def _refill():
        pltpu.make_async_copy(
            w1_hbm.at[pl.ds(nxt * ST, ST)], w1buf.at[slot],
            sems.at[slot, 0]).start()
        pltpu.make_async_copy(
            w2_hbm.at[pl.ds(nxt * ST, ST)], w2buf.at[slot],
            sems.at[slot, 1]).start()


@functools.partial(jax.jit)
def kernel(x, Wg, bg, W1, b1, W2, b2, gamma, beta):
    comp = pl.pallas_call(
        _route_kernel,
        grid=(B, NB),
        in_specs=[
            pl.BlockSpec((1, CN, D), lambda b, n: (b, n, 0)),
            pl.BlockSpec((D, S), lambda b, n: (0, 0)),
            pl.BlockSpec((S,), lambda b, n: (0,)),
        ],
        out_specs=pl.BlockSpec((1, S, D), lambda b, n: (b, 0, 0)),
        out_shape=jax.ShapeDtypeStruct((B, S, D), jnp.float32),
        scratch_shapes=[pltpu.VMEM((1, S), jnp.float32)],
        compiler_params=pltpu.CompilerParams(
            dimension_semantics=("parallel", "arbitrary")),
    )(x, Wg, bg)

    compT = comp.transpose(1, 0, 2)
    final = pl.pallas_call(
        _mlp_kernel,
        grid=(NSTEP,),
        in_specs=[
            pl.BlockSpec((S, B, D), lambda s: (0, 0, 0)),
            pl.BlockSpec((S, 1, H), lambda s: (0, 0, 0)),
            pl.BlockSpec((S, 1, D), lambda s: (0, 0, 0)),
            pl.BlockSpec((S, 1, D), lambda s: (0, 0, 0)),
            pl.BlockSpec((S, 1, D), lambda s: (0, 0, 0)),
            pl.BlockSpec(memory_space=pltpu.HBM),
            pl.BlockSpec(memory_space=pltpu.HBM),
        ],
        out_specs=pl.BlockSpec((ST, B, D), lambda s: (s, 0, 0)),
        out_shape=jax.ShapeDtypeStruct((S, B, D), jnp.float32),
        scratch_shapes=[
            pltpu.VMEM((NBUF, ST, D, H), jnp.float32),
            pltpu.VMEM((NBUF, ST, H, D), jnp.float32),
            pltpu.SemaphoreType.DMA((NBUF, 2)),
        ],
        compiler_params=pltpu.CompilerParams(
            dimension_semantics=("arbitrary",),
            vmem_limit_bytes=100 * 1024 * 1024),
    )(compT, b1[:, None, :], b2[:, None, :], gamma[:, None, :],
      beta[:, None, :], W1, W2).transpose(1, 0, 2)

    aux_loss = jnp.array(0.0, dtype=jnp.float32)
    return (final, aux_loss)


# speedup vs baseline: 1.0954x; 1.0954x over previous
import functools

import jax
import jax.numpy as jnp
from jax.experimental import pallas as pl
from jax.experimental.pallas import tpu as pltpu

B, N, D = 2, 4096, 1024
S, K, H = 64, 2, 512

CN = 1024
NB = N // CN
RSTEPS = B * NB
ST = 2
NSTEP = S // ST
NBUF = 6


def _fused_kernel(x_ref, wg_ref, bg_ref, b1_ref, b2_ref, g_ref, be_ref,
                  w1_hbm, w2_hbm, out_ref,
                  comp_ref, cnt_ref, w1buf, w2buf, sems):
    t = pl.program_id(0)

    for k in range(NBUF):
        @pl.when(t == k)
        def _prefetch(k=k):
            pltpu.make_async_copy(
                w1_hbm.at[pl.ds(k * ST, ST)], w1buf.at[k],
                sems.at[k, 0]).start()
            pltpu.make_async_copy(
                w2_hbm.at[pl.ds(k * ST, ST)], w2buf.at[k],
                sems.at[k, 1]).start()

    @pl.when(t < RSTEPS)
    def _route():
        xc = x_ref[0]
        logits = jax.lax.dot_general(
            xc, wg_ref[...], (((1,), (0,)), ((), ())),
            preferred_element_type=jnp.float32) + bg_ref[...]
        m1 = jnp.max(logits, axis=-1, keepdims=True)
        l2 = jnp.where(logits >= m1, -jnp.inf, logits)
        m2 = jnp.max(l2, axis=-1, keepdims=True)
        e = jnp.exp(logits - m1)
        sum_e = jnp.sum(e, axis=-1, keepdims=True)
        w = jnp.where(logits >= m2, e, 0.0) / sum_e

        part = jax.lax.dot_general(
            w, xc, (((0,), (0,)), ((), ())),
            preferred_element_type=jnp.float32)
        csum = jnp.sum(w, axis=0)[None, :]

        for b in range(B):
            base = b * NB

            @pl.when((t >= base) & (t < base + NB))
            def _batch(b=b, base=base):
                @pl.when(t == base)
                def _init():
                    comp_ref[:, b, :] = part
                    cnt_ref[...] = csum

                @pl.when(t > base)
                def _acc():
                    comp_ref[:, b, :] += part
                    cnt_ref[...] += csum

                @pl.when(t == base + NB - 1)
                def _final():
                    comp_ref[:, b, :] = (
                        comp_ref[:, b, :] / (cnt_ref[0][:, None] + 1e-9))

    @pl.when(t >= RSTEPS)
    def _mlp():
        i = t - RSTEPS
        slot = jax.lax.rem(i, NBUF)
        pltpu.make_async_copy(
            w1_hbm.at[pl.ds(i * ST, ST)], w1buf.at[slot],
            sems.at[slot, 0]).wait()
        pltpu.make_async_copy(
            w2_hbm.at[pl.ds(i * ST, ST)], w2buf.at[slot],
            sems.at[slot, 1]).wait()

        for j in range(ST):
            idx = i * ST + j
            c = comp_ref[idx]
            h = jax.lax.dot_general(
                c, w1buf[slot, j], (((1,), (0,)), ((), ())),
                preferred_element_type=jnp.float32) + b1_ref[idx]
            h = 0.5 * h * (1.0 + jax.lax.erf(h * 0.7071067811865476))
            y = jax.lax.dot_general(
                h, w2buf[slot, j], (((1,), (0,)), ((), ())),
                preferred_element_type=jnp.float32) + b2_ref[idx]
            mu = jnp.mean(y, axis=-1, keepdims=True)
            var = jnp.mean((y - mu) ** 2, axis=-1, keepdims=True)
            ln = (y - mu) * jax.lax.rsqrt(var + 1e-5) * g_ref[idx] + be_ref[idx]
            out_ref[j] = c + ln

        nxt = i + NBUF

        @pl.when(nxt < NSTEP)
        def _refill():
            pltpu.make_async_copy(
                w1_hbm.at[pl.ds(nxt * ST, ST)], w1buf.at[slot],
                sems.at[slot, 0]).start()
            pltpu.make_async_copy(
                w2_hbm.at[pl.ds(nxt * ST, ST)], w2buf.at[slot],
                sems.at[slot, 1]).start()


@functools.partial(jax.jit)
def kernel(x, Wg, bg, W1, b1, W2, b2, gamma, beta):
    final = pl.pallas_call(
        _fused_kernel,
        grid=(RSTEPS + NSTEP,),
        in_specs=[
            pl.BlockSpec(
                (1, CN, D),
                lambda t: (jnp.minimum(t // NB, B - 1),
                           jnp.where(t < RSTEPS, jax.lax.rem(t, NB), NB - 1),
                           0)),
            pl.BlockSpec((D, S), lambda t: (0, 0)),
            pl.BlockSpec((S,), lambda t: (0,)),
            pl.BlockSpec((S, 1, H), lambda t: (0, 0, 0)),
            pl.BlockSpec((S, 1, D), lambda t: (0, 0, 0)),
            pl.BlockSpec((S, 1, D), lambda t: (0, 0, 0)),
            pl.BlockSpec((S, 1, D), lambda t: (0, 0, 0)),
            pl.BlockSpec(memory_space=pltpu.HBM),
            pl.BlockSpec(memory_space=pltpu.HBM),
        ],
        out_specs=pl.BlockSpec(
            (ST, B, D),
            lambda t: (jnp.where(t < RSTEPS, 0, t - RSTEPS), 0, 0)),
        out_shape=jax.ShapeDtypeStruct((S, B, D), jnp.float32),
        scratch_shapes=[
            pltpu.VMEM((S, B, D), jnp.float32),
            pltpu.VMEM((1, S), jnp.float32),
            pltpu.VMEM((NBUF, ST, D, H), jnp.float32),
            pltpu.VMEM((NBUF, ST, H, D), jnp.float32),
            pltpu.SemaphoreType.DMA((NBUF, 2)),
        ],
        compiler_params=pltpu.CompilerParams(
            dimension_semantics=("arbitrary",),
            vmem_limit_bytes=64 * 1024 * 1024),
    )(x, Wg, bg, b1[:, None, :], b2[:, None, :],
      gamma[:, None, :], beta[:, None, :], W1, W2).transpose(1, 0, 2)

    aux_loss = jnp.array(0.0, dtype=jnp.float32)
    return (final, aux_loss)
